# Initial kernel scaffold; baseline (speedup 1.0000x reference)
import jax
import jax.numpy as jnp
from jax.experimental import pallas as pl


def kernel(x, edge_index, W_enc1, b_enc1, W_class, b_class, W_dec1, b_dec1, W_xbar, b_xbar, W_g1, b_g1, W_g2, b_g2, W_pnd, b_pnd):
    raise NotImplementedError("write your pallas kernel here")


# speedup vs baseline: 28.2333x; 28.2333x over previous
import functools

import jax
import jax.numpy as jnp
from jax import lax
from jax.experimental import pallas as pl
from jax.experimental.pallas import tpu as pltpu
from jax.experimental.pallas import tpu_sc as plsc

F32 = jnp.float32

_N = 10000
_E = 320000
_B = 125
_NC = 2
_NS = 16
_NW = _NC * _NS
_RPW = _E // (_B * _NW)
_RPT = _N // _NS
_BLK = 1000


@functools.cache
def _make_agg(D, gather):
  mesh = plsc.VectorSubcoreMesh(core_axis_name="c", subcore_axis_name="s")
  scratch = []
  if gather:
    scratch.append(pltpu.VMEM((_RPW, _B), jnp.int32))
  scratch += [
      pltpu.VMEM((_RPW, _B), jnp.int32),
      pltpu.VMEM((_B, D), F32),
      pltpu.VMEM_SHARED((_N, D), F32),
      pltpu.SemaphoreType.DMA,
  ]

  def body(*refs):
    if gather:
      (vals_hbm, src_hbm, dst_hbm, out_hbm,
       src_v, dst_v, rows_v, acc, sem) = refs
    else:
      dst_hbm, out_hbm, dst_v, rows_v, acc, sem = refs
    cid = lax.axis_index("c")
    tid = lax.axis_index("s")
    wid = tid * _NC + cid

    def zero_row(r, carry):
      for j in range(D // 16):
        rows_v[r, pl.ds(j * 16, 16)] = jnp.zeros((16,), F32)
      return carry
    lax.fori_loop(0, _B, zero_row, 0)
    for k in range(_RPT // _B):
      pltpu.sync_copy(rows_v, acc.at[pl.ds(tid * _RPT + k * _B, _B)])
    plsc.subcore_barrier()

    pltpu.sync_copy(dst_hbm.at[pl.ds(wid * _RPW, _RPW)], dst_v)
    if gather:
      pltpu.sync_copy(src_hbm.at[pl.ds(wid * _RPW, _RPW)], src_v)
    else:
      def ones_row(r, carry):
        for j in range(D // 16):
          rows_v[r, pl.ds(j * 16, 16)] = jnp.ones((16,), F32)
        return carry
      lax.fori_loop(0, _B, ones_row, 0)

    def chunk(c, carry):
      if gather:
        pltpu.async_copy(vals_hbm.at[src_v.at[c]], rows_v, sem).wait()
      pltpu.sync_copy(rows_v, acc.at[dst_v.at[c]], add=True)
      return carry
    lax.fori_loop(0, _RPW, chunk, 0)
    plsc.subcore_barrier()

    pltpu.sync_copy(acc.at[pl.ds(tid * _RPT, _RPT)],
                    out_hbm.at[cid, pl.ds(tid * _RPT, _RPT)])

  return functools.partial(
      pl.kernel,
      mesh=mesh,
      out_type=jax.ShapeDtypeStruct((_NC, _N, D), F32),
      scratch_types=scratch,
      compiler_params=pltpu.CompilerParams(use_tc_tiling_on_sc=False),
  )(body)


def _row_spec(d):
  return pl.BlockSpec((_BLK, d), lambda i: (i, 0))


def _part_spec(d):
  return pl.BlockSpec((_NC, _BLK, d), lambda i: (0, i, 0))


def _full_spec(a, b):
  return pl.BlockSpec((a, b), lambda i: (0, 0))


def _tc1_body(x_r, dg_r, we1, be1, wc, bc, wd1, bd1, wxb, bxb,
              xs_o, enc_o, z_o, xbar_o, dis_o):
  x = x_r[...]
  deg = dg_r[0, :, 0:1] + dg_r[1, :, 0:1] + 1.0
  dis = lax.rsqrt(deg)
  dis_o[...] = dis
  xs_o[...] = x * dis
  enc = jnp.maximum(jnp.dot(x, we1[...], preferred_element_type=F32) + be1[...], 0.0)
  enc_o[...] = enc
  z = jnp.dot(enc, wc[...], preferred_element_type=F32) + bc[...]
  z_o[...] = z
  dec = jnp.maximum(jnp.dot(z, wd1[...], preferred_element_type=F32) + bd1[...], 0.0)
  xbar_o[...] = jnp.dot(dec, wxb[...], preferred_element_type=F32) + bxb[...]


def _tc1(x, degp, W_enc1, b_enc1, W_class, b_class, W_dec1, b_dec1, W_xbar, b_xbar):
  return pl.pallas_call(
      _tc1_body,
      grid=(_N // _BLK,),
      in_specs=[
          _row_spec(128), _part_spec(16),
          _full_spec(128, 256), _full_spec(1, 256),
          _full_spec(256, 64), _full_spec(1, 64),
          _full_spec(64, 256), _full_spec(1, 256),
          _full_spec(256, 128), _full_spec(1, 128),
      ],
      out_specs=[_row_spec(128), _row_spec(256), _row_spec(64),
                 _row_spec(128), _row_spec(1)],
      out_shape=[
          jax.ShapeDtypeStruct((_N, 128), F32),
          jax.ShapeDtypeStruct((_N, 256), F32),
          jax.ShapeDtypeStruct((_N, 64), F32),
          jax.ShapeDtypeStruct((_N, 128), F32),
          jax.ShapeDtypeStruct((_N, 1), F32),
      ],
  )(x, degp, W_enc1, b_enc1, W_class, b_class, W_dec1, b_dec1, W_xbar, b_xbar)


def _tc2_body(acc_r, x_r, dis_r, enc_r, z_r, wg1, bg1, wg2, wpnd,
              m2_o, m3_o, mc_o):
  dis = dis_r[...]
  a = acc_r[0] + acc_r[1]
  out1 = dis * a + (dis * dis) * x_r[...]
  h1 = jnp.dot(out1, wg1[...], preferred_element_type=F32) + bg1[...]
  u = 0.5 * h1 + 0.5 * enc_r[...]
  m2 = jnp.dot(u, wg2[...], preferred_element_type=F32)
  m3 = jnp.dot(z_r[...], wpnd[...], preferred_element_type=F32)
  m2_o[...] = m2
  m3_o[...] = m3
  mc_o[...] = jnp.concatenate(
      [m2 * dis, m3 * dis, jnp.zeros((_BLK, 15), F32)], axis=1)


def _tc2(acc1, x, dis, enc, z, W_g1, b_g1, W_g2, W_pnd):
  return pl.pallas_call(
      _tc2_body,
      grid=(_N // _BLK,),
      in_specs=[
          _part_spec(128), _row_spec(128), _row_spec(1),
          _row_spec(256), _row_spec(64),
          _full_spec(128, 256), _full_spec(1, 256),
          _full_spec(256, 16), _full_spec(64, 1),
      ],
      out_specs=[_row_spec(16), _row_spec(1), _row_spec(32)],
      out_shape=[
          jax.ShapeDtypeStruct((_N, 16), F32),
          jax.ShapeDtypeStruct((_N, 1), F32),
          jax.ShapeDtypeStruct((_N, 32), F32),
      ],
  )(acc1, x, dis, enc, z, W_g1, b_g1, W_g2, W_pnd)


def _tc3_body(acc_r, m2_r, m3_r, dis_r, bg2, bpnd, pred_o, hp_o):
  dis = dis_r[...]
  d2 = dis * dis
  a = acc_r[0] + acc_r[1]
  h2 = dis * a[:, 0:16] + d2 * m2_r[...] + bg2[...]
  mx = jnp.max(h2, axis=1, keepdims=True)
  e = h2 - mx
  lse = jnp.log(jnp.sum(jnp.exp(e), axis=1, keepdims=True))
  pred_o[...] = e - lse
  hp_o[...] = dis * a[:, 16:17] + d2 * m3_r[...] + bpnd[...]


def _tc3(acc2, m2, m3, dis, b_g2, b_pnd):
  return pl.pallas_call(
      _tc3_body,
      grid=(_N // _BLK,),
      in_specs=[
          _part_spec(32), _row_spec(16), _row_spec(1), _row_spec(1),
          _full_spec(1, 16), _full_spec(1, 1),
      ],
      out_specs=[_row_spec(16), _row_spec(1)],
      out_shape=[
          jax.ShapeDtypeStruct((_N, 16), F32),
          jax.ShapeDtypeStruct((_N, 1), F32),
      ],
  )(acc2, m2, m3, dis, b_g2, b_pnd)


def kernel(x, edge_index, W_enc1, b_enc1, W_class, b_class, W_dec1, b_dec1,
           W_xbar, b_xbar, W_g1, b_g1, W_g2, b_g2, W_pnd, b_pnd):
  ei = edge_index.astype(jnp.int32)
  src2d = ei[0].reshape(_E // _B, _B)
  dst2d = ei[1].reshape(_E // _B, _B)

  degp = _make_agg(16, gather=False)(dst2d)
  xs, enc, z, x_bar, dis = _tc1(
      x, degp, W_enc1, b_enc1.reshape(1, -1), W_class, b_class.reshape(1, -1),
      W_dec1, b_dec1.reshape(1, -1), W_xbar, b_xbar.reshape(1, -1))
  acc1 = _make_agg(128, gather=True)(xs, src2d, dst2d)
  m2, m3, mc = _tc2(acc1, x, dis, enc, z, W_g1, b_g1.reshape(1, -1), W_g2, W_pnd)
  acc2 = _make_agg(32, gather=True)(mc, src2d, dst2d)
  predict, h_pred_nd = _tc3(acc2, m2, m3, dis,
                            b_g2.reshape(1, -1), b_pnd.reshape(1, -1))
  return (x_bar, predict, h_pred_nd)


# speedup vs baseline: 33.1698x; 1.1748x over previous
import functools

import jax
import jax.numpy as jnp
from jax import lax
from jax.experimental import pallas as pl
from jax.experimental.pallas import tpu as pltpu
from jax.experimental.pallas import tpu_sc as plsc

F32 = jnp.float32

_N = 10000
_E = 320000
_B = 125
_NC = 2
_NS = 16
_NW = _NC * _NS
_RPW = _E // (_B * _NW)
_RPT = _N // _NS
_BLK = 1000


@functools.cache
def _make_agg(D, gather):
  mesh = plsc.VectorSubcoreMesh(core_axis_name="c", subcore_axis_name="s")
  nsec = 2 if gather else 1
  srpw = _RPW // nsec
  scratch = []
  if gather:
    scratch.append(pltpu.VMEM((srpw, _B), jnp.int32))
  scratch += [
      pltpu.VMEM((srpw, _B), jnp.int32),
      pltpu.VMEM((_B, D), F32),
      pltpu.VMEM((_B, D), F32),
      pltpu.VMEM_SHARED((_N, D), F32),
      pltpu.SemaphoreType.DMA,
      pltpu.SemaphoreType.DMA,
  ]

  def body(*refs):
    if gather:
      (vals_hbm, src_hbm, dst_hbm, out_hbm,
       src_v, dst_v, rows_v, rows1_v, acc, sem, sem1) = refs
    else:
      dst_hbm, out_hbm, dst_v, rows_v, rows1_v, acc, sem, sem1 = refs
    cid = lax.axis_index("c")
    tid = lax.axis_index("s")
    wid = tid * _NC + cid

    def zero_row(r, carry):
      for j in range(D // 16):
        rows_v[r, pl.ds(j * 16, 16)] = jnp.zeros((16,), F32)
      return carry
    lax.fori_loop(0, _B, zero_row, 0)
    for k in range(_RPT // _B):
      pltpu.sync_copy(rows_v, acc.at[pl.ds(tid * _RPT + k * _B, _B)])
    plsc.subcore_barrier()

    if gather:
      for s in range(nsec):
        base = wid * _RPW + s * srpw
        pltpu.sync_copy(dst_hbm.at[pl.ds(base, srpw)], dst_v)
        pltpu.sync_copy(src_hbm.at[pl.ds(base, srpw)], src_v)
        pltpu.async_copy(vals_hbm.at[src_v.at[0]], rows_v, sem)

        def chunk2(c2, carry):
          c = c2 * 2
          pltpu.make_async_copy(vals_hbm.at[src_v.at[c]], rows_v, sem).wait()
          pltpu.async_copy(vals_hbm.at[src_v.at[c + 1]], rows1_v, sem1)
          pltpu.sync_copy(rows_v, acc.at[dst_v.at[c]], add=True)
          pltpu.make_async_copy(
              vals_hbm.at[src_v.at[c + 1]], rows1_v, sem1).wait()

          @pl.when(c + 2 < srpw)
          def _():
            pltpu.async_copy(vals_hbm.at[src_v.at[c + 2]], rows_v, sem)
          pltpu.sync_copy(rows1_v, acc.at[dst_v.at[c + 1]], add=True)
          return carry
        lax.fori_loop(0, srpw // 2, chunk2, 0)
    else:
      pltpu.sync_copy(dst_hbm.at[pl.ds(wid * _RPW, _RPW)], dst_v)

      def ones_row(r, carry):
        for j in range(D // 16):
          rows_v[r, pl.ds(j * 16, 16)] = jnp.ones((16,), F32)
        return carry
      lax.fori_loop(0, _B, ones_row, 0)

      def chunk(c, carry):
        pltpu.sync_copy(rows_v, acc.at[dst_v.at[c]], add=True)
        return carry
      lax.fori_loop(0, _RPW, chunk, 0)
    plsc.subcore_barrier()

    pltpu.sync_copy(acc.at[pl.ds(tid * _RPT, _RPT)],
                    out_hbm.at[cid, pl.ds(tid * _RPT, _RPT)])

  return functools.partial(
      pl.kernel,
      mesh=mesh,
      out_type=jax.ShapeDtypeStruct((_NC, _N, D), F32),
      scratch_types=scratch,
      compiler_params=pltpu.CompilerParams(use_tc_tiling_on_sc=False),
  )(body)


def _row_spec(d):
  return pl.BlockSpec((_BLK, d), lambda i: (i, 0))


def _part_spec(d):
  return pl.BlockSpec((_NC, _BLK, d), lambda i: (0, i, 0))


def _full_spec(a, b):
  return pl.BlockSpec((a, b), lambda i: (0, 0))


def _tc1_body(x_r, dg_r, we1, be1, wc, bc, wd1, bd1, wxb, bxb,
              xs_o, enc_o, z_o, xbar_o, dis_o):
  x = x_r[...]
  deg = dg_r[0, :, 0:1] + dg_r[1, :, 0:1] + 1.0
  dis = lax.rsqrt(deg)
  dis_o[...] = dis
  xs_o[...] = x * dis
  enc = jnp.maximum(jnp.dot(x, we1[...], preferred_element_type=F32) + be1[...], 0.0)
  enc_o[...] = enc
  z = jnp.dot(enc, wc[...], preferred_element_type=F32) + bc[...]
  z_o[...] = z
  dec = jnp.maximum(jnp.dot(z, wd1[...], preferred_element_type=F32) + bd1[...], 0.0)
  xbar_o[...] = jnp.dot(dec, wxb[...], preferred_element_type=F32) + bxb[...]


def _tc1(x, degp, W_enc1, b_enc1, W_class, b_class, W_dec1, b_dec1, W_xbar, b_xbar):
  return pl.pallas_call(
      _tc1_body,
      grid=(_N // _BLK,),
      in_specs=[
          _row_spec(128), _part_spec(16),
          _full_spec(128, 256), _full_spec(1, 256),
          _full_spec(256, 64), _full_spec(1, 64),
          _full_spec(64, 256), _full_spec(1, 256),
          _full_spec(256, 128), _full_spec(1, 128),
      ],
      out_specs=[_row_spec(128), _row_spec(256), _row_spec(64),
                 _row_spec(128), _row_spec(1)],
      out_shape=[
          jax.ShapeDtypeStruct((_N, 128), F32),
          jax.ShapeDtypeStruct((_N, 256), F32),
          jax.ShapeDtypeStruct((_N, 64), F32),
          jax.ShapeDtypeStruct((_N, 128), F32),
          jax.ShapeDtypeStruct((_N, 1), F32),
      ],
  )(x, degp, W_enc1, b_enc1, W_class, b_class, W_dec1, b_dec1, W_xbar, b_xbar)


def _tc2_body(acc_r, x_r, dis_r, enc_r, z_r, wg1, bg1, wg2, wpnd,
              m2_o, m3_o, mc_o):
  dis = dis_r[...]
  a = acc_r[0] + acc_r[1]
  out1 = dis * a + (dis * dis) * x_r[...]
  h1 = jnp.dot(out1, wg1[...], preferred_element_type=F32) + bg1[...]
  u = 0.5 * h1 + 0.5 * enc_r[...]
  m2 = jnp.dot(u, wg2[...], preferred_element_type=F32)
  m3 = jnp.dot(z_r[...], wpnd[...], preferred_element_type=F32)
  m2_o[...] = m2
  m3_o[...] = m3
  mc_o[...] = jnp.concatenate(
      [m2 * dis, m3 * dis, jnp.zeros((_BLK, 15), F32)], axis=1)


def _tc2(acc1, x, dis, enc, z, W_g1, b_g1, W_g2, W_pnd):
  return pl.pallas_call(
      _tc2_body,
      grid=(_N // _BLK,),
      in_specs=[
          _part_spec(128), _row_spec(128), _row_spec(1),
          _row_spec(256), _row_spec(64),
          _full_spec(128, 256), _full_spec(1, 256),
          _full_spec(256, 16), _full_spec(64, 1),
      ],
      out_specs=[_row_spec(16), _row_spec(1), _row_spec(32)],
      out_shape=[
          jax.ShapeDtypeStruct((_N, 16), F32),
          jax.ShapeDtypeStruct((_N, 1), F32),
          jax.ShapeDtypeStruct((_N, 32), F32),
      ],
  )(acc1, x, dis, enc, z, W_g1, b_g1, W_g2, W_pnd)


def _tc3_body(acc_r, m2_r, m3_r, dis_r, bg2, bpnd, pred_o, hp_o):
  dis = dis_r[...]
  d2 = dis * dis
  a = acc_r[0] + acc_r[1]
  h2 = dis * a[:, 0:16] + d2 * m2_r[...] + bg2[...]
  mx = jnp.max(h2, axis=1, keepdims=True)
  e = h2 - mx
  lse = jnp.log(jnp.sum(jnp.exp(e), axis=1, keepdims=True))
  pred_o[...] = e - lse
  hp_o[...] = dis * a[:, 16:17] + d2 * m3_r[...] + bpnd[...]


def _tc3(acc2, m2, m3, dis, b_g2, b_pnd):
  return pl.pallas_call(
      _tc3_body,
      grid=(_N // _BLK,),
      in_specs=[
          _part_spec(32), _row_spec(16), _row_spec(1), _row_spec(1),
          _full_spec(1, 16), _full_spec(1, 1),
      ],
      out_specs=[_row_spec(16), _row_spec(1)],
      out_shape=[
          jax.ShapeDtypeStruct((_N, 16), F32),
          jax.ShapeDtypeStruct((_N, 1), F32),
      ],
  )(acc2, m2, m3, dis, b_g2, b_pnd)


def kernel(x, edge_index, W_enc1, b_enc1, W_class, b_class, W_dec1, b_dec1,
           W_xbar, b_xbar, W_g1, b_g1, W_g2, b_g2, W_pnd, b_pnd):
  ei = edge_index.astype(jnp.int32)
  src2d = ei[0].reshape(_E // _B, _B)
  dst2d = ei[1].reshape(_E // _B, _B)

  degp = _make_agg(16, gather=False)(dst2d)
  xs, enc, z, x_bar, dis = _tc1(
      x, degp, W_enc1, b_enc1.reshape(1, -1), W_class, b_class.reshape(1, -1),
      W_dec1, b_dec1.reshape(1, -1), W_xbar, b_xbar.reshape(1, -1))
  acc1 = _make_agg(128, gather=True)(xs, src2d, dst2d)
  m2, m3, mc = _tc2(acc1, x, dis, enc, z, W_g1, b_g1.reshape(1, -1), W_g2, W_pnd)
  acc2 = _make_agg(32, gather=True)(mc, src2d, dst2d)
  predict, h_pred_nd = _tc3(acc2, m2, m3, dis,
                            b_g2.reshape(1, -1), b_pnd.reshape(1, -1))
  return (x_bar, predict, h_pred_nd)


# speedup vs baseline: 33.7459x; 1.0174x over previous
import functools

import jax
import jax.numpy as jnp
from jax import lax
from jax.experimental import pallas as pl
from jax.experimental.pallas import tpu as pltpu
from jax.experimental.pallas import tpu_sc as plsc

F32 = jnp.float32

_N = 10000
_E = 320000
_B = 125
_NC = 2
_NS = 16
_NW = _NC * _NS
_RPW = _E // (_B * _NW)
_RPT = _N // _NS
_BLK = 1000


@functools.cache
def _make_agg(D, gather):
  mesh = plsc.VectorSubcoreMesh(core_axis_name="c", subcore_axis_name="s")
  nsec = 2 if gather else 1
  srpw = _RPW // nsec
  scratch = []
  if gather:
    scratch.append(pltpu.VMEM((srpw, _B), jnp.int32))
  scratch += [
      pltpu.VMEM((srpw, _B), jnp.int32),
      pltpu.VMEM((_B, D), F32),
      pltpu.VMEM((_B, D), F32),
      pltpu.VMEM_SHARED((_N, D), F32),
      pltpu.SemaphoreType.DMA,
      pltpu.SemaphoreType.DMA,
      pltpu.SemaphoreType.DMA,
      pltpu.SemaphoreType.DMA,
  ]

  def body(*refs):
    if gather:
      (vals_hbm, src_hbm, dst_hbm, out_hbm,
       src_v, dst_v, rows_v, rows1_v, acc, sem, sem1, sem2, sem3) = refs
    else:
      (dst_hbm, out_hbm, dst_v, rows_v, rows1_v, acc,
       sem, sem1, sem2, sem3) = refs
    cid = lax.axis_index("c")
    tid = lax.axis_index("s")
    wid = tid * _NC + cid

    def zero_row(r, carry):
      for j in range(D // 16):
        rows_v[r, pl.ds(j * 16, 16)] = jnp.zeros((16,), F32)
      return carry
    lax.fori_loop(0, _B, zero_row, 0)
    for k in range(_RPT // _B):
      pltpu.sync_copy(rows_v, acc.at[pl.ds(tid * _RPT + k * _B, _B)])
    plsc.subcore_barrier()

    if gather:
      for s in range(nsec):
        base = wid * _RPW + s * srpw
        pltpu.sync_copy(dst_hbm.at[pl.ds(base, srpw)], dst_v)
        pltpu.sync_copy(src_hbm.at[pl.ds(base, srpw)], src_v)
        pltpu.async_copy(vals_hbm.at[src_v.at[0]], rows_v, sem)
        pltpu.async_copy(vals_hbm.at[src_v.at[1]], rows1_v, sem1)

        def chunk2(c2, carry):
          c = c2 * 2
          pltpu.make_async_copy(vals_hbm.at[src_v.at[c]], rows_v, sem).wait()
          pltpu.async_copy(rows_v, acc.at[dst_v.at[c]], sem2, add=True)
          pltpu.make_async_copy(
              vals_hbm.at[src_v.at[c + 1]], rows1_v, sem1).wait()
          pltpu.async_copy(rows1_v, acc.at[dst_v.at[c + 1]], sem3, add=True)
          pltpu.make_async_copy(rows_v, acc.at[dst_v.at[c]], sem2).wait()

          @pl.when(c + 2 < srpw)
          def _():
            pltpu.async_copy(vals_hbm.at[src_v.at[c + 2]], rows_v, sem)
          pltpu.make_async_copy(rows1_v, acc.at[dst_v.at[c + 1]], sem3).wait()

          @pl.when(c + 3 < srpw)
          def _():
            pltpu.async_copy(vals_hbm.at[src_v.at[c + 3]], rows1_v, sem1)
          return carry
        lax.fori_loop(0, srpw // 2, chunk2, 0)
    else:
      pltpu.sync_copy(dst_hbm.at[pl.ds(wid * _RPW, _RPW)], dst_v)

      def ones_row(r, carry):
        for j in range(D // 16):
          rows_v[r, pl.ds(j * 16, 16)] = jnp.ones((16,), F32)
        return carry
      lax.fori_loop(0, _B, ones_row, 0)

      def batch(b, carry):
        for k in range(8):
          pltpu.async_copy(rows_v, acc.at[dst_v.at[b * 8 + k]], sem, add=True)
        for k in range(8):
          pltpu.make_async_copy(rows_v, acc.at[dst_v.at[b * 8 + k]], sem).wait()
        return carry
      lax.fori_loop(0, _RPW // 8, batch, 0)
    plsc.subcore_barrier()

    pltpu.sync_copy(acc.at[pl.ds(tid * _RPT, _RPT)],
                    out_hbm.at[cid, pl.ds(tid * _RPT, _RPT)])

  return functools.partial(
      pl.kernel,
      mesh=mesh,
      out_type=jax.ShapeDtypeStruct((_NC, _N, D), F32),
      scratch_types=scratch,
      compiler_params=pltpu.CompilerParams(use_tc_tiling_on_sc=False),
  )(body)


def _row_spec(d):
  return pl.BlockSpec((_BLK, d), lambda i: (i, 0))


def _part_spec(d):
  return pl.BlockSpec((_NC, _BLK, d), lambda i: (0, i, 0))


def _full_spec(a, b):
  return pl.BlockSpec((a, b), lambda i: (0, 0))


def _tc1_body(x_r, dg_r, we1, be1, wc, bc, wd1, bd1, wxb, bxb,
              xs_o, enc_o, z_o, xbar_o, dis_o):
  x = x_r[...]
  deg = dg_r[0, :, 0:1] + dg_r[1, :, 0:1] + 1.0
  dis = lax.rsqrt(deg)
  dis_o[...] = dis
  xs_o[...] = x * dis
  enc = jnp.maximum(jnp.dot(x, we1[...], preferred_element_type=F32) + be1[...], 0.0)
  enc_o[...] = enc
  z = jnp.dot(enc, wc[...], preferred_element_type=F32) + bc[...]
  z_o[...] = z
  dec = jnp.maximum(jnp.dot(z, wd1[...], preferred_element_type=F32) + bd1[...], 0.0)
  xbar_o[...] = jnp.dot(dec, wxb[...], preferred_element_type=F32) + bxb[...]


def _tc1(x, degp, W_enc1, b_enc1, W_class, b_class, W_dec1, b_dec1, W_xbar, b_xbar):
  return pl.pallas_call(
      _tc1_body,
      grid=(_N // _BLK,),
      in_specs=[
          _row_spec(128), _part_spec(16),
          _full_spec(128, 256), _full_spec(1, 256),
          _full_spec(256, 64), _full_spec(1, 64),
          _full_spec(64, 256), _full_spec(1, 256),
          _full_spec(256, 128), _full_spec(1, 128),
      ],
      out_specs=[_row_spec(128), _row_spec(256), _row_spec(64),
                 _row_spec(128), _row_spec(1)],
      out_shape=[
          jax.ShapeDtypeStruct((_N, 128), F32),
          jax.ShapeDtypeStruct((_N, 256), F32),
          jax.ShapeDtypeStruct((_N, 64), F32),
          jax.ShapeDtypeStruct((_N, 128), F32),
          jax.ShapeDtypeStruct((_N, 1), F32),
      ],
  )(x, degp, W_enc1, b_enc1, W_class, b_class, W_dec1, b_dec1, W_xbar, b_xbar)


def _tc2_body(acc_r, x_r, dis_r, enc_r, z_r, wg1, bg1, wg2, wpnd,
              m2_o, m3_o, mc_o):
  dis = dis_r[...]
  a = acc_r[0] + acc_r[1]
  out1 = dis * a + (dis * dis) * x_r[...]
  h1 = jnp.dot(out1, wg1[...], preferred_element_type=F32) + bg1[...]
  u = 0.5 * h1 + 0.5 * enc_r[...]
  m2 = jnp.dot(u, wg2[...], preferred_element_type=F32)
  m3 = jnp.dot(z_r[...], wpnd[...], preferred_element_type=F32)
  m2_o[...] = m2
  m3_o[...] = m3
  mc_o[...] = jnp.concatenate(
      [m2 * dis, m3 * dis, jnp.zeros((_BLK, 15), F32)], axis=1)


def _tc2(acc1, x, dis, enc, z, W_g1, b_g1, W_g2, W_pnd):
  return pl.pallas_call(
      _tc2_body,
      grid=(_N // _BLK,),
      in_specs=[
          _part_spec(128), _row_spec(128), _row_spec(1),
          _row_spec(256), _row_spec(64),
          _full_spec(128, 256), _full_spec(1, 256),
          _full_spec(256, 16), _full_spec(64, 1),
      ],
      out_specs=[_row_spec(16), _row_spec(1), _row_spec(32)],
      out_shape=[
          jax.ShapeDtypeStruct((_N, 16), F32),
          jax.ShapeDtypeStruct((_N, 1), F32),
          jax.ShapeDtypeStruct((_N, 32), F32),
      ],
  )(acc1, x, dis, enc, z, W_g1, b_g1, W_g2, W_pnd)


def _tc3_body(acc_r, m2_r, m3_r, dis_r, bg2, bpnd, pred_o, hp_o):
  dis = dis_r[...]
  d2 = dis * dis
  a = acc_r[0] + acc_r[1]
  h2 = dis * a[:, 0:16] + d2 * m2_r[...] + bg2[...]
  mx = jnp.max(h2, axis=1, keepdims=True)
  e = h2 - mx
  lse = jnp.log(jnp.sum(jnp.exp(e), axis=1, keepdims=True))
  pred_o[...] = e - lse
  hp_o[...] = dis * a[:, 16:17] + d2 * m3_r[...] + bpnd[...]


def _tc3(acc2, m2, m3, dis, b_g2, b_pnd):
  return pl.pallas_call(
      _tc3_body,
      grid=(_N // _BLK,),
      in_specs=[
          _part_spec(32), _row_spec(16), _row_spec(1), _row_spec(1),
          _full_spec(1, 16), _full_spec(1, 1),
      ],
      out_specs=[_row_spec(16), _row_spec(1)],
      out_shape=[
          jax.ShapeDtypeStruct((_N, 16), F32),
          jax.ShapeDtypeStruct((_N, 1), F32),
      ],
  )(acc2, m2, m3, dis, b_g2, b_pnd)


def kernel(x, edge_index, W_enc1, b_enc1, W_class, b_class, W_dec1, b_dec1,
           W_xbar, b_xbar, W_g1, b_g1, W_g2, b_g2, W_pnd, b_pnd):
  ei = edge_index.astype(jnp.int32)
  src2d = ei[0].reshape(_E // _B, _B)
  dst2d = ei[1].reshape(_E // _B, _B)

  degp = _make_agg(16, gather=False)(dst2d)
  xs, enc, z, x_bar, dis = _tc1(
      x, degp, W_enc1, b_enc1.reshape(1, -1), W_class, b_class.reshape(1, -1),
      W_dec1, b_dec1.reshape(1, -1), W_xbar, b_xbar.reshape(1, -1))
  acc1 = _make_agg(128, gather=True)(xs, src2d, dst2d)
  m2, m3, mc = _tc2(acc1, x, dis, enc, z, W_g1, b_g1.reshape(1, -1), W_g2, W_pnd)
  acc2 = _make_agg(32, gather=True)(mc, src2d, dst2d)
  predict, h_pred_nd = _tc3(acc2, m2, m3, dis,
                            b_g2.reshape(1, -1), b_pnd.reshape(1, -1))
  return (x_bar, predict, h_pred_nd)


# speedup vs baseline: 37.7712x; 1.1193x over previous
import functools

import jax
import jax.numpy as jnp
from jax import lax
from jax.experimental import pallas as pl
from jax.experimental.pallas import tpu as pltpu
from jax.experimental.pallas import tpu_sc as plsc

F32 = jnp.float32

_N = 10000
_E = 320000
_B = 125
_NC = 2
_NS = 16
_NW = _NC * _NS
_RPW = _E // (_B * _NW)
_RPT = _N // _NS
_BLK = 1000


@functools.cache
def _make_agg(D, gather, dtype=F32):
  lanes = 32 if dtype == jnp.bfloat16 else 16
  mesh = plsc.VectorSubcoreMesh(core_axis_name="c", subcore_axis_name="s")
  nsec = 2 if gather else 1
  srpw = _RPW // nsec
  scratch = []
  if gather:
    scratch.append(pltpu.VMEM((srpw, _B), jnp.int32))
  scratch += [
      pltpu.VMEM((srpw, _B), jnp.int32),
      pltpu.VMEM((_B, D), dtype),
      pltpu.VMEM((_B, D), dtype),
      pltpu.VMEM_SHARED((_N, D), dtype),
      pltpu.SemaphoreType.DMA,
      pltpu.SemaphoreType.DMA,
      pltpu.SemaphoreType.DMA,
      pltpu.SemaphoreType.DMA,
  ]

  def body(*refs):
    if gather:
      (vals_hbm, src_hbm, dst_hbm, out_hbm,
       src_v, dst_v, rows_v, rows1_v, acc, sem, sem1, sem2, sem3) = refs
    else:
      (dst_hbm, out_hbm, dst_v, rows_v, rows1_v, acc,
       sem, sem1, sem2, sem3) = refs
    cid = lax.axis_index("c")
    tid = lax.axis_index("s")
    wid = tid * _NC + cid

    def zero_row(r, carry):
      for j in range(D // lanes):
        rows_v[r, pl.ds(j * lanes, lanes)] = jnp.zeros((lanes,), dtype)
      return carry
    lax.fori_loop(0, _B, zero_row, 0)
    for k in range(_RPT // _B):
      pltpu.sync_copy(rows_v, acc.at[pl.ds(tid * _RPT + k * _B, _B)])
    plsc.subcore_barrier()

    if gather:
      for s in range(nsec):
        base = wid * _RPW + s * srpw
        pltpu.sync_copy(dst_hbm.at[pl.ds(base, srpw)], dst_v)
        pltpu.sync_copy(src_hbm.at[pl.ds(base, srpw)], src_v)
        pltpu.async_copy(vals_hbm.at[src_v.at[0]], rows_v, sem)
        pltpu.async_copy(vals_hbm.at[src_v.at[1]], rows1_v, sem1)

        def chunk2(c2, carry):
          c = c2 * 2
          pltpu.make_async_copy(vals_hbm.at[src_v.at[c]], rows_v, sem).wait()
          pltpu.async_copy(rows_v, acc.at[dst_v.at[c]], sem2, add=True)
          pltpu.make_async_copy(
              vals_hbm.at[src_v.at[c + 1]], rows1_v, sem1).wait()
          pltpu.async_copy(rows1_v, acc.at[dst_v.at[c + 1]], sem3, add=True)
          pltpu.make_async_copy(rows_v, acc.at[dst_v.at[c]], sem2).wait()

          @pl.when(c + 2 < srpw)
          def _():
            pltpu.async_copy(vals_hbm.at[src_v.at[c + 2]], rows_v, sem)
          pltpu.make_async_copy(rows1_v, acc.at[dst_v.at[c + 1]], sem3).wait()

          @pl.when(c + 3 < srpw)
          def _():
            pltpu.async_copy(vals_hbm.at[src_v.at[c + 3]], rows1_v, sem1)
          return carry
        lax.fori_loop(0, srpw // 2, chunk2, 0)
    else:
      pltpu.sync_copy(dst_hbm.at[pl.ds(wid * _RPW, _RPW)], dst_v)

      def ones_row(r, carry):
        for j in range(D // lanes):
          rows_v[r, pl.ds(j * lanes, lanes)] = jnp.ones((lanes,), dtype)
        return carry
      lax.fori_loop(0, _B, ones_row, 0)

      def batch(b, carry):
        for k in range(8):
          pltpu.async_copy(rows_v, acc.at[dst_v.at[b * 8 + k]], sem, add=True)
        for k in range(8):
          pltpu.make_async_copy(rows_v, acc.at[dst_v.at[b * 8 + k]], sem).wait()
        return carry
      lax.fori_loop(0, _RPW // 8, batch, 0)
    plsc.subcore_barrier()

    pltpu.sync_copy(acc.at[pl.ds(tid * _RPT, _RPT)],
                    out_hbm.at[cid, pl.ds(tid * _RPT, _RPT)])

  return functools.partial(
      pl.kernel,
      mesh=mesh,
      out_type=jax.ShapeDtypeStruct((_NC, _N, D), dtype),
      scratch_types=scratch,
      compiler_params=pltpu.CompilerParams(use_tc_tiling_on_sc=False),
  )(body)


def _row_spec(d):
  return pl.BlockSpec((_BLK, d), lambda i: (i, 0))


def _part_spec(d):
  return pl.BlockSpec((_NC, _BLK, d), lambda i: (0, i, 0))


def _full_spec(a, b):
  return pl.BlockSpec((a, b), lambda i: (0, 0))


def _tc1_body(x_r, dg_r, we1, be1, wc, bc, wd1, bd1, wxb, bxb,
              xs_o, enc_o, z_o, xbar_o, dis_o):
  x = x_r[...]
  deg = dg_r[0, :, 0:1] + dg_r[1, :, 0:1] + 1.0
  dis = lax.rsqrt(deg)
  dis_o[...] = dis
  xs_o[...] = (x * dis).astype(jnp.bfloat16)
  enc = jnp.maximum(jnp.dot(x, we1[...], preferred_element_type=F32) + be1[...], 0.0)
  enc_o[...] = enc
  z = jnp.dot(enc, wc[...], preferred_element_type=F32) + bc[...]
  z_o[...] = z
  dec = jnp.maximum(jnp.dot(z, wd1[...], preferred_element_type=F32) + bd1[...], 0.0)
  xbar_o[...] = jnp.dot(dec, wxb[...], preferred_element_type=F32) + bxb[...]


def _tc1(x, degp, W_enc1, b_enc1, W_class, b_class, W_dec1, b_dec1, W_xbar, b_xbar):
  return pl.pallas_call(
      _tc1_body,
      grid=(_N // _BLK,),
      in_specs=[
          _row_spec(128), _part_spec(16),
          _full_spec(128, 256), _full_spec(1, 256),
          _full_spec(256, 64), _full_spec(1, 64),
          _full_spec(64, 256), _full_spec(1, 256),
          _full_spec(256, 128), _full_spec(1, 128),
      ],
      out_specs=[_row_spec(128), _row_spec(256), _row_spec(64),
                 _row_spec(128), _row_spec(1)],
      out_shape=[
          jax.ShapeDtypeStruct((_N, 128), jnp.bfloat16),
          jax.ShapeDtypeStruct((_N, 256), F32),
          jax.ShapeDtypeStruct((_N, 64), F32),
          jax.ShapeDtypeStruct((_N, 128), F32),
          jax.ShapeDtypeStruct((_N, 1), F32),
      ],
  )(x, degp, W_enc1, b_enc1, W_class, b_class, W_dec1, b_dec1, W_xbar, b_xbar)


def _tc2_body(acc_r, x_r, dis_r, enc_r, z_r, wg1, bg1, wg2, wpnd,
              m2_o, m3_o, mc_o):
  dis = dis_r[...]
  a = acc_r[0].astype(F32) + acc_r[1].astype(F32)
  out1 = dis * a + (dis * dis) * x_r[...]
  h1 = jnp.dot(out1, wg1[...], preferred_element_type=F32) + bg1[...]
  u = 0.5 * h1 + 0.5 * enc_r[...]
  m2 = jnp.dot(u, wg2[...], preferred_element_type=F32)
  m3 = jnp.dot(z_r[...], wpnd[...], preferred_element_type=F32)
  m2_o[...] = m2
  m3_o[...] = m3
  mc_o[...] = jnp.concatenate(
      [m2 * dis, m3 * dis, jnp.zeros((_BLK, 15), F32)], axis=1
  ).astype(jnp.bfloat16)


def _tc2(acc1, x, dis, enc, z, W_g1, b_g1, W_g2, W_pnd):
  return pl.pallas_call(
      _tc2_body,
      grid=(_N // _BLK,),
      in_specs=[
          _part_spec(128), _row_spec(128), _row_spec(1),
          _row_spec(256), _row_spec(64),
          _full_spec(128, 256), _full_spec(1, 256),
          _full_spec(256, 16), _full_spec(64, 1),
      ],
      out_specs=[_row_spec(16), _row_spec(1), _row_spec(32)],
      out_shape=[
          jax.ShapeDtypeStruct((_N, 16), F32),
          jax.ShapeDtypeStruct((_N, 1), F32),
          jax.ShapeDtypeStruct((_N, 32), jnp.bfloat16),
      ],
  )(acc1, x, dis, enc, z, W_g1, b_g1, W_g2, W_pnd)


def _tc3_body(acc_r, m2_r, m3_r, dis_r, bg2, bpnd, pred_o, hp_o):
  dis = dis_r[...]
  d2 = dis * dis
  a = acc_r[0].astype(F32) + acc_r[1].astype(F32)
  h2 = dis * a[:, 0:16] + d2 * m2_r[...] + bg2[...]
  mx = jnp.max(h2, axis=1, keepdims=True)
  e = h2 - mx
  lse = jnp.log(jnp.sum(jnp.exp(e), axis=1, keepdims=True))
  pred_o[...] = e - lse
  hp_o[...] = dis * a[:, 16:17] + d2 * m3_r[...] + bpnd[...]


def _tc3(acc2, m2, m3, dis, b_g2, b_pnd):
  return pl.pallas_call(
      _tc3_body,
      grid=(_N // _BLK,),
      in_specs=[
          _part_spec(32), _row_spec(16), _row_spec(1), _row_spec(1),
          _full_spec(1, 16), _full_spec(1, 1),
      ],
      out_specs=[_row_spec(16), _row_spec(1)],
      out_shape=[
          jax.ShapeDtypeStruct((_N, 16), F32),
          jax.ShapeDtypeStruct((_N, 1), F32),
      ],
  )(acc2, m2, m3, dis, b_g2, b_pnd)


def kernel(x, edge_index, W_enc1, b_enc1, W_class, b_class, W_dec1, b_dec1,
           W_xbar, b_xbar, W_g1, b_g1, W_g2, b_g2, W_pnd, b_pnd):
  ei = edge_index.astype(jnp.int32)
  src2d = ei[0].reshape(_E // _B, _B)
  dst2d = ei[1].reshape(_E // _B, _B)

  degp = _make_agg(16, gather=False)(dst2d)
  xs, enc, z, x_bar, dis = _tc1(
      x, degp, W_enc1, b_enc1.reshape(1, -1), W_class, b_class.reshape(1, -1),
      W_dec1, b_dec1.reshape(1, -1), W_xbar, b_xbar.reshape(1, -1))
  acc1 = _make_agg(128, gather=True, dtype=jnp.bfloat16)(xs, src2d, dst2d)
  m2, m3, mc = _tc2(acc1, x, dis, enc, z, W_g1, b_g1.reshape(1, -1), W_g2, W_pnd)
  acc2 = _make_agg(32, gather=True, dtype=jnp.bfloat16)(mc, src2d, dst2d)
  predict, h_pred_nd = _tc3(acc2, m2, m3, dis,
                            b_g2.reshape(1, -1), b_pnd.reshape(1, -1))
  return (x_bar, predict, h_pred_nd)


# speedup vs baseline: 43.5144x; 1.1521x over previous
import functools

import jax
import jax.numpy as jnp
from jax import lax
from jax.experimental import pallas as pl
from jax.experimental.pallas import tpu as pltpu
from jax.experimental.pallas import tpu_sc as plsc

F32 = jnp.float32

_N = 10000
_E = 320000
_B = 125
_NC = 2
_NS = 16
_NW = _NC * _NS
_RPW = _E // (_B * _NW)
_RPT = _N // _NS
_BLK = 1000


@functools.cache
def _make_agg(D, gather, dtype=F32):
  lanes = 32 if dtype == jnp.bfloat16 else 16
  mesh = plsc.VectorSubcoreMesh(core_axis_name="c", subcore_axis_name="s")
  nsec = 2 if gather else 1
  srpw = _RPW // nsec
  scratch = []
  nbuf = 4
  if gather:
    scratch.append(pltpu.VMEM((srpw, _B), jnp.int32))
  scratch += [
      pltpu.VMEM((srpw, _B), jnp.int32),
  ]
  scratch += [pltpu.VMEM((_B, D), dtype)] * nbuf
  scratch += [pltpu.VMEM_SHARED((_N, D), dtype)]
  scratch += [pltpu.SemaphoreType.DMA] * (2 * nbuf)

  def body(*refs):
    if gather:
      (vals_hbm, src_hbm, dst_hbm, out_hbm, src_v, dst_v) = refs[:6]
      rest = refs[6:]
    else:
      (dst_hbm, out_hbm, dst_v) = refs[:3]
      rest = refs[3:]
    rows = rest[:nbuf]
    acc = rest[nbuf]
    gsem = rest[nbuf + 1:nbuf + 1 + nbuf]
    ssem = rest[nbuf + 1 + nbuf:]
    rows_v = rows[0]
    sem = gsem[0]
    cid = lax.axis_index("c")
    tid = lax.axis_index("s")
    wid = tid * _NC + cid

    def zero_row(r, carry):
      for j in range(D // lanes):
        rows_v[r, pl.ds(j * lanes, lanes)] = jnp.zeros((lanes,), dtype)
      return carry
    lax.fori_loop(0, _B, zero_row, 0)
    for k in range(_RPT // _B):
      pltpu.sync_copy(rows_v, acc.at[pl.ds(tid * _RPT + k * _B, _B)])
    plsc.subcore_barrier()

    if gather:
      for s in range(nsec):
        base = wid * _RPW + s * srpw
        pltpu.sync_copy(dst_hbm.at[pl.ds(base, srpw)], dst_v)
        pltpu.sync_copy(src_hbm.at[pl.ds(base, srpw)], src_v)
        for i in range(nbuf):
          pltpu.async_copy(vals_hbm.at[src_v.at[i]], rows[i], gsem[i])

        def chunkn(k, carry):
          c = k * nbuf
          for i in range(nbuf):
            pltpu.make_async_copy(
                vals_hbm.at[src_v.at[c + i]], rows[i], gsem[i]).wait()
            pltpu.async_copy(rows[i], acc.at[dst_v.at[c + i]], ssem[i],
                             add=True)
          for i in range(nbuf):
            pltpu.make_async_copy(
                rows[i], acc.at[dst_v.at[c + i]], ssem[i]).wait()

            @pl.when(c + nbuf + i < srpw)
            def _():
              pltpu.async_copy(
                  vals_hbm.at[src_v.at[c + nbuf + i]], rows[i], gsem[i])
          return carry
        lax.fori_loop(0, srpw // nbuf, chunkn, 0)
    else:
      pltpu.sync_copy(dst_hbm.at[pl.ds(wid * _RPW, _RPW)], dst_v)

      def ones_row(r, carry):
        for j in range(D // lanes):
          rows_v[r, pl.ds(j * lanes, lanes)] = jnp.ones((lanes,), dtype)
        return carry
      lax.fori_loop(0, _B, ones_row, 0)

      def batch(b, carry):
        for k in range(8):
          pltpu.async_copy(rows_v, acc.at[dst_v.at[b * 8 + k]], sem, add=True)
        for k in range(8):
          pltpu.make_async_copy(rows_v, acc.at[dst_v.at[b * 8 + k]], sem).wait()
        return carry
      lax.fori_loop(0, _RPW // 8, batch, 0)
    plsc.subcore_barrier()

    pltpu.sync_copy(acc.at[pl.ds(tid * _RPT, _RPT)],
                    out_hbm.at[cid, pl.ds(tid * _RPT, _RPT)])

  return functools.partial(
      pl.kernel,
      mesh=mesh,
      out_type=jax.ShapeDtypeStruct((_NC, _N, D), dtype),
      scratch_types=scratch,
      compiler_params=pltpu.CompilerParams(use_tc_tiling_on_sc=False),
  )(body)


def _row_spec(d):
  return pl.BlockSpec((_BLK, d), lambda i: (i, 0))


def _part_spec(d):
  return pl.BlockSpec((_NC, _BLK, d), lambda i: (0, i, 0))


def _full_spec(a, b):
  return pl.BlockSpec((a, b), lambda i: (0, 0))


def _tc1_body(x_r, dg_r, we1, be1, wc, bc, wd1, bd1, wxb, bxb,
              xs_o, enc_o, z_o, xbar_o, dis_o):
  x = x_r[...]
  deg = dg_r[0, :, 0:1] + dg_r[1, :, 0:1] + 1.0
  dis = lax.rsqrt(deg)
  dis_o[...] = dis
  xs_o[...] = (x * dis).astype(jnp.bfloat16)
  enc = jnp.maximum(jnp.dot(x, we1[...], preferred_element_type=F32) + be1[...], 0.0)
  enc_o[...] = enc
  z = jnp.dot(enc, wc[...], preferred_element_type=F32) + bc[...]
  z_o[...] = z
  dec = jnp.maximum(jnp.dot(z, wd1[...], preferred_element_type=F32) + bd1[...], 0.0)
  xbar_o[...] = jnp.dot(dec, wxb[...], preferred_element_type=F32) + bxb[...]


def _tc1(x, degp, W_enc1, b_enc1, W_class, b_class, W_dec1, b_dec1, W_xbar, b_xbar):
  return pl.pallas_call(
      _tc1_body,
      grid=(_N // _BLK,),
      in_specs=[
          _row_spec(128), _part_spec(16),
          _full_spec(128, 256), _full_spec(1, 256),
          _full_spec(256, 64), _full_spec(1, 64),
          _full_spec(64, 256), _full_spec(1, 256),
          _full_spec(256, 128), _full_spec(1, 128),
      ],
      out_specs=[_row_spec(128), _row_spec(256), _row_spec(64),
                 _row_spec(128), _row_spec(1)],
      out_shape=[
          jax.ShapeDtypeStruct((_N, 128), jnp.bfloat16),
          jax.ShapeDtypeStruct((_N, 256), F32),
          jax.ShapeDtypeStruct((_N, 64), F32),
          jax.ShapeDtypeStruct((_N, 128), F32),
          jax.ShapeDtypeStruct((_N, 1), F32),
      ],
  )(x, degp, W_enc1, b_enc1, W_class, b_class, W_dec1, b_dec1, W_xbar, b_xbar)


def _tc2_body(acc_r, x_r, dis_r, enc_r, z_r, wg1, bg1, wg2, wpnd,
              m2_o, m3_o, mc_o):
  dis = dis_r[...]
  a = acc_r[0].astype(F32) + acc_r[1].astype(F32)
  out1 = dis * a + (dis * dis) * x_r[...]
  h1 = jnp.dot(out1, wg1[...], preferred_element_type=F32) + bg1[...]
  u = 0.5 * h1 + 0.5 * enc_r[...]
  m2 = jnp.dot(u, wg2[...], preferred_element_type=F32)
  m3 = jnp.dot(z_r[...], wpnd[...], preferred_element_type=F32)
  m2_o[...] = m2
  m3_o[...] = m3
  mc_o[...] = jnp.concatenate(
      [m2 * dis, m3 * dis, jnp.zeros((_BLK, 15), F32)], axis=1
  ).astype(jnp.bfloat16)


def _tc2(acc1, x, dis, enc, z, W_g1, b_g1, W_g2, W_pnd):
  return pl.pallas_call(
      _tc2_body,
      grid=(_N // _BLK,),
      in_specs=[
          _part_spec(128), _row_spec(128), _row_spec(1),
          _row_spec(256), _row_spec(64),
          _full_spec(128, 256), _full_spec(1, 256),
          _full_spec(256, 16), _full_spec(64, 1),
      ],
      out_specs=[_row_spec(16), _row_spec(1), _row_spec(32)],
      out_shape=[
          jax.ShapeDtypeStruct((_N, 16), F32),
          jax.ShapeDtypeStruct((_N, 1), F32),
          jax.ShapeDtypeStruct((_N, 32), jnp.bfloat16),
      ],
  )(acc1, x, dis, enc, z, W_g1, b_g1, W_g2, W_pnd)


def _tc3_body(acc_r, m2_r, m3_r, dis_r, bg2, bpnd, pred_o, hp_o):
  dis = dis_r[...]
  d2 = dis * dis
  a = acc_r[0].astype(F32) + acc_r[1].astype(F32)
  h2 = dis * a[:, 0:16] + d2 * m2_r[...] + bg2[...]
  mx = jnp.max(h2, axis=1, keepdims=True)
  e = h2 - mx
  lse = jnp.log(jnp.sum(jnp.exp(e), axis=1, keepdims=True))
  pred_o[...] = e - lse
  hp_o[...] = dis * a[:, 16:17] + d2 * m3_r[...] + bpnd[...]


def _tc3(acc2, m2, m3, dis, b_g2, b_pnd):
  return pl.pallas_call(
      _tc3_body,
      grid=(_N // _BLK,),
      in_specs=[
          _part_spec(32), _row_spec(16), _row_spec(1), _row_spec(1),
          _full_spec(1, 16), _full_spec(1, 1),
      ],
      out_specs=[_row_spec(16), _row_spec(1)],
      out_shape=[
          jax.ShapeDtypeStruct((_N, 16), F32),
          jax.ShapeDtypeStruct((_N, 1), F32),
      ],
  )(acc2, m2, m3, dis, b_g2, b_pnd)


def kernel(x, edge_index, W_enc1, b_enc1, W_class, b_class, W_dec1, b_dec1,
           W_xbar, b_xbar, W_g1, b_g1, W_g2, b_g2, W_pnd, b_pnd):
  ei = edge_index.astype(jnp.int32)
  src2d = ei[0].reshape(_E // _B, _B)
  dst2d = ei[1].reshape(_E // _B, _B)

  degp = _make_agg(16, gather=False)(dst2d)
  xs, enc, z, x_bar, dis = _tc1(
      x, degp, W_enc1, b_enc1.reshape(1, -1), W_class, b_class.reshape(1, -1),
      W_dec1, b_dec1.reshape(1, -1), W_xbar, b_xbar.reshape(1, -1))
  acc1 = _make_agg(128, gather=True, dtype=jnp.bfloat16)(xs, src2d, dst2d)
  m2, m3, mc = _tc2(acc1, x, dis, enc, z, W_g1, b_g1.reshape(1, -1), W_g2, W_pnd)
  acc2 = _make_agg(32, gather=True, dtype=jnp.bfloat16)(mc, src2d, dst2d)
  predict, h_pred_nd = _tc3(acc2, m2, m3, dis,
                            b_g2.reshape(1, -1), b_pnd.reshape(1, -1))
  return (x_bar, predict, h_pred_nd)


# speedup vs baseline: 44.2361x; 1.0166x over previous
import functools

import jax
import jax.numpy as jnp
from jax import lax
from jax.experimental import pallas as pl
from jax.experimental.pallas import tpu as pltpu
from jax.experimental.pallas import tpu_sc as plsc

F32 = jnp.float32

_N = 10000
_E = 320000
_B = 125
_NC = 2
_NS = 16
_NW = _NC * _NS
_RPW = _E // (_B * _NW)
_RPT = _N // _NS
_BLK = 1000


@functools.cache
def _make_agg(D, gather, dtype=F32):
  lanes = 32 if dtype == jnp.bfloat16 else 16
  mesh = plsc.VectorSubcoreMesh(core_axis_name="c", subcore_axis_name="s")
  nsec = 2 if gather else 1
  srpw = _RPW // nsec
  scratch = []
  nbuf = 4
  if gather:
    scratch.append(pltpu.VMEM((srpw, _B), jnp.int32))
  scratch += [
      pltpu.VMEM((srpw, _B), jnp.int32),
  ]
  scratch += [pltpu.VMEM((_B, D), dtype)] * nbuf
  scratch += [pltpu.VMEM_SHARED((_N, D), dtype)]
  scratch += [pltpu.SemaphoreType.DMA] * (2 * nbuf)

  def body(*refs):
    if gather:
      (vals_hbm, src_hbm, dst_hbm, out_hbm, src_v, dst_v) = refs[:6]
      rest = refs[6:]
    else:
      (dst_hbm, out_hbm, dst_v) = refs[:3]
      rest = refs[3:]
    rows = rest[:nbuf]
    acc = rest[nbuf]
    gsem = rest[nbuf + 1:nbuf + 1 + nbuf]
    ssem = rest[nbuf + 1 + nbuf:]
    rows_v = rows[0]
    sem = gsem[0]
    cid = lax.axis_index("c")
    tid = lax.axis_index("s")
    wid = tid * _NC + cid

    def zero_row(r, carry):
      for j in range(D // lanes):
        rows_v[r, pl.ds(j * lanes, lanes)] = jnp.zeros((lanes,), dtype)
      return carry
    lax.fori_loop(0, _B, zero_row, 0)
    for k in range(_RPT // _B):
      pltpu.sync_copy(rows_v, acc.at[pl.ds(tid * _RPT + k * _B, _B)])
    plsc.subcore_barrier()

    if gather:
      for s in range(nsec):
        base = wid * _RPW + s * srpw
        pltpu.sync_copy(dst_hbm.at[pl.ds(base, srpw)], dst_v)
        pltpu.sync_copy(src_hbm.at[pl.ds(base, srpw)], src_v)
        for i in range(nbuf):
          pltpu.async_copy(vals_hbm.at[src_v.at[i]], rows[i], gsem[i])

        def chunkn(k, carry):
          c = k * nbuf
          for i in range(nbuf):
            pltpu.make_async_copy(
                vals_hbm.at[src_v.at[c + i]], rows[i], gsem[i]).wait()
            pltpu.async_copy(rows[i], acc.at[dst_v.at[c + i]], ssem[i],
                             add=True)
          for i in range(nbuf):
            pltpu.make_async_copy(
                rows[i], acc.at[dst_v.at[c + i]], ssem[i]).wait()

            @pl.when(c + nbuf + i < srpw)
            def _():
              pltpu.async_copy(
                  vals_hbm.at[src_v.at[c + nbuf + i]], rows[i], gsem[i])
          return carry
        lax.fori_loop(0, srpw // nbuf, chunkn, 0)
    else:
      pltpu.sync_copy(dst_hbm.at[pl.ds(wid * _RPW, _RPW)], dst_v)

      def ones_row(r, carry):
        for j in range(D // lanes):
          rows_v[r, pl.ds(j * lanes, lanes)] = jnp.ones((lanes,), dtype)
        return carry
      lax.fori_loop(0, _B, ones_row, 0)

      def batch(b, carry):
        for k in range(8):
          pltpu.async_copy(rows_v, acc.at[dst_v.at[b * 8 + k]], sem, add=True)
        for k in range(8):
          pltpu.make_async_copy(rows_v, acc.at[dst_v.at[b * 8 + k]], sem).wait()
        return carry
      lax.fori_loop(0, _RPW // 8, batch, 0)
    plsc.subcore_barrier()

    pltpu.sync_copy(acc.at[pl.ds(tid * _RPT, _RPT)],
                    out_hbm.at[cid, pl.ds(tid * _RPT, _RPT)])

  return functools.partial(
      pl.kernel,
      mesh=mesh,
      out_type=jax.ShapeDtypeStruct((_NC, _N, D), dtype),
      scratch_types=scratch,
      compiler_params=pltpu.CompilerParams(use_tc_tiling_on_sc=False),
  )(body)


def _row_spec(d):
  return pl.BlockSpec((_BLK, d), lambda i: (i, 0))


def _part_spec(d):
  return pl.BlockSpec((_NC, _BLK, d), lambda i: (0, i, 0))


def _full_spec(a, b):
  return pl.BlockSpec((a, b), lambda i: (0, 0))


def _tc0_body(x_r, we1, be1, wc, bc, wd1, bd1, wxb, bxb,
              enc_o, z_o, xbar_o):
  x = x_r[...]
  enc = jnp.maximum(jnp.dot(x, we1[...], preferred_element_type=F32) + be1[...], 0.0)
  enc_o[...] = enc
  z = jnp.dot(enc, wc[...], preferred_element_type=F32) + bc[...]
  z_o[...] = z
  dec = jnp.maximum(jnp.dot(z, wd1[...], preferred_element_type=F32) + bd1[...], 0.0)
  xbar_o[...] = jnp.dot(dec, wxb[...], preferred_element_type=F32) + bxb[...]


def _tc0(x, W_enc1, b_enc1, W_class, b_class, W_dec1, b_dec1, W_xbar, b_xbar):
  return pl.pallas_call(
      _tc0_body,
      grid=(_N // _BLK,),
      in_specs=[
          _row_spec(128),
          _full_spec(128, 256), _full_spec(1, 256),
          _full_spec(256, 64), _full_spec(1, 64),
          _full_spec(64, 256), _full_spec(1, 256),
          _full_spec(256, 128), _full_spec(1, 128),
      ],
      out_specs=[_row_spec(256), _row_spec(64), _row_spec(128)],
      out_shape=[
          jax.ShapeDtypeStruct((_N, 256), F32),
          jax.ShapeDtypeStruct((_N, 64), F32),
          jax.ShapeDtypeStruct((_N, 128), F32),
      ],
  )(x, W_enc1, b_enc1, W_class, b_class, W_dec1, b_dec1, W_xbar, b_xbar)


def _tc1_body(x_r, dg_r, xs_o, dis_o):
  x = x_r[...]
  deg = dg_r[0, :, 0:1] + dg_r[1, :, 0:1] + 1.0
  dis = lax.rsqrt(deg)
  dis_o[...] = dis
  xs_o[...] = (x * dis).astype(jnp.bfloat16)


def _tc1(x, degp):
  return pl.pallas_call(
      _tc1_body,
      grid=(_N // _BLK,),
      in_specs=[_row_spec(128), _part_spec(16)],
      out_specs=[_row_spec(128), _row_spec(1)],
      out_shape=[
          jax.ShapeDtypeStruct((_N, 128), jnp.bfloat16),
          jax.ShapeDtypeStruct((_N, 1), F32),
      ],
  )(x, degp)


def _tc2_body(acc_r, x_r, dis_r, enc_r, z_r, wg1, bg1, wg2, wpnd,
              m2_o, m3_o, mc_o):
  dis = dis_r[...]
  a = acc_r[0].astype(F32) + acc_r[1].astype(F32)
  out1 = dis * a + (dis * dis) * x_r[...]
  h1 = jnp.dot(out1, wg1[...], preferred_element_type=F32) + bg1[...]
  u = 0.5 * h1 + 0.5 * enc_r[...]
  m2 = jnp.dot(u, wg2[...], preferred_element_type=F32)
  m3 = jnp.dot(z_r[...], wpnd[...], preferred_element_type=F32)
  m2_o[...] = m2
  m3_o[...] = m3
  mc_o[...] = jnp.concatenate(
      [m2 * dis, m3 * dis, jnp.zeros((_BLK, 15), F32)], axis=1
  ).astype(jnp.bfloat16)


def _tc2(acc1, x, dis, enc, z, W_g1, b_g1, W_g2, W_pnd):
  return pl.pallas_call(
      _tc2_body,
      grid=(_N // _BLK,),
      in_specs=[
          _part_spec(128), _row_spec(128), _row_spec(1),
          _row_spec(256), _row_spec(64),
          _full_spec(128, 256), _full_spec(1, 256),
          _full_spec(256, 16), _full_spec(64, 1),
      ],
      out_specs=[_row_spec(16), _row_spec(1), _row_spec(32)],
      out_shape=[
          jax.ShapeDtypeStruct((_N, 16), F32),
          jax.ShapeDtypeStruct((_N, 1), F32),
          jax.ShapeDtypeStruct((_N, 32), jnp.bfloat16),
      ],
  )(acc1, x, dis, enc, z, W_g1, b_g1, W_g2, W_pnd)


def _tc3_body(acc_r, m2_r, m3_r, dis_r, bg2, bpnd, pred_o, hp_o):
  dis = dis_r[...]
  d2 = dis * dis
  a = acc_r[0].astype(F32) + acc_r[1].astype(F32)
  h2 = dis * a[:, 0:16] + d2 * m2_r[...] + bg2[...]
  mx = jnp.max(h2, axis=1, keepdims=True)
  e = h2 - mx
  lse = jnp.log(jnp.sum(jnp.exp(e), axis=1, keepdims=True))
  pred_o[...] = e - lse
  hp_o[...] = dis * a[:, 16:17] + d2 * m3_r[...] + bpnd[...]


def _tc3(acc2, m2, m3, dis, b_g2, b_pnd):
  return pl.pallas_call(
      _tc3_body,
      grid=(_N // _BLK,),
      in_specs=[
          _part_spec(32), _row_spec(16), _row_spec(1), _row_spec(1),
          _full_spec(1, 16), _full_spec(1, 1),
      ],
      out_specs=[_row_spec(16), _row_spec(1)],
      out_shape=[
          jax.ShapeDtypeStruct((_N, 16), F32),
          jax.ShapeDtypeStruct((_N, 1), F32),
      ],
  )(acc2, m2, m3, dis, b_g2, b_pnd)


def kernel(x, edge_index, W_enc1, b_enc1, W_class, b_class, W_dec1, b_dec1,
           W_xbar, b_xbar, W_g1, b_g1, W_g2, b_g2, W_pnd, b_pnd):
  ei = edge_index.astype(jnp.int32)
  src2d = ei[0].reshape(_E // _B, _B)
  dst2d = ei[1].reshape(_E // _B, _B)

  degp = _make_agg(16, gather=False)(dst2d)
  enc, z, x_bar = _tc0(
      x, W_enc1, b_enc1.reshape(1, -1), W_class, b_class.reshape(1, -1),
      W_dec1, b_dec1.reshape(1, -1), W_xbar, b_xbar.reshape(1, -1))
  xs, dis = _tc1(x, degp)
  acc1 = _make_agg(128, gather=True, dtype=jnp.bfloat16)(xs, src2d, dst2d)
  m2, m3, mc = _tc2(acc1, x, dis, enc, z, W_g1, b_g1.reshape(1, -1), W_g2, W_pnd)
  acc2 = _make_agg(32, gather=True, dtype=jnp.bfloat16)(mc, src2d, dst2d)
  predict, h_pred_nd = _tc3(acc2, m2, m3, dis,
                            b_g2.reshape(1, -1), b_pnd.reshape(1, -1))
  return (x_bar, predict, h_pred_nd)


# speedup vs baseline: 45.6048x; 1.0309x over previous
import functools

import jax
import jax.numpy as jnp
from jax import lax
from jax.experimental import pallas as pl
from jax.experimental.pallas import tpu as pltpu
from jax.experimental.pallas import tpu_sc as plsc

F32 = jnp.float32

_N = 10000
_E = 320000
_B = 125
_NC = 2
_NS = 16
_NW = _NC * _NS
_RPW = _E // (_B * _NW)
_RPT = _N // _NS
_BLK = 1000


@functools.cache
def _make_agg(D, gather, dtype=F32):
  lanes = 32 if dtype == jnp.bfloat16 else 16
  mesh = plsc.VectorSubcoreMesh(core_axis_name="c", subcore_axis_name="s")
  nsec = 2 if gather else 1
  srpw = _RPW // nsec
  scratch = []
  nbuf = 8 if gather else 4
  if gather:
    scratch.append(pltpu.VMEM((srpw, _B), jnp.int32))
  scratch += [
      pltpu.VMEM((srpw, _B), jnp.int32),
  ]
  scratch += [pltpu.VMEM((_B, D), dtype)] * nbuf
  scratch += [pltpu.VMEM_SHARED((_N, D), dtype)]
  scratch += [pltpu.SemaphoreType.DMA] * (2 * nbuf)

  def body(*refs):
    if gather:
      (vals_hbm, src_hbm, dst_hbm, out_hbm, src_v, dst_v) = refs[:6]
      rest = refs[6:]
    else:
      (dst_hbm, out_hbm, dst_v) = refs[:3]
      rest = refs[3:]
    rows = rest[:nbuf]
    acc = rest[nbuf]
    gsem = rest[nbuf + 1:nbuf + 1 + nbuf]
    ssem = rest[nbuf + 1 + nbuf:]
    rows_v = rows[0]
    sem = gsem[0]
    cid = lax.axis_index("c")
    tid = lax.axis_index("s")
    wid = tid * _NC + cid

    def zero_row(r, carry):
      for j in range(D // lanes):
        rows_v[r, pl.ds(j * lanes, lanes)] = jnp.zeros((lanes,), dtype)
      return carry
    lax.fori_loop(0, _B, zero_row, 0)
    for k in range(_RPT // _B):
      pltpu.sync_copy(rows_v, acc.at[pl.ds(tid * _RPT + k * _B, _B)])
    plsc.subcore_barrier()

    if gather:
      for s in range(nsec):
        base = wid * _RPW + s * srpw
        pltpu.sync_copy(dst_hbm.at[pl.ds(base, srpw)], dst_v)
        pltpu.sync_copy(src_hbm.at[pl.ds(base, srpw)], src_v)
        for i in range(nbuf):
          pltpu.async_copy(vals_hbm.at[src_v.at[i]], rows[i], gsem[i])

        def chunkn(k, carry):
          c = k * nbuf
          for i in range(nbuf):
            pltpu.make_async_copy(
                vals_hbm.at[src_v.at[c + i]], rows[i], gsem[i]).wait()
            pltpu.async_copy(rows[i], acc.at[dst_v.at[c + i]], ssem[i],
                             add=True)
          for i in range(nbuf):
            pltpu.make_async_copy(
                rows[i], acc.at[dst_v.at[c + i]], ssem[i]).wait()

            @pl.when(c + nbuf + i < srpw)
            def _():
              pltpu.async_copy(
                  vals_hbm.at[src_v.at[c + nbuf + i]], rows[i], gsem[i])
          return carry
        lax.fori_loop(0, srpw // nbuf, chunkn, 0)
    else:
      pltpu.sync_copy(dst_hbm.at[pl.ds(wid * _RPW, _RPW)], dst_v)

      def ones_row(r, carry):
        for j in range(D // lanes):
          rows_v[r, pl.ds(j * lanes, lanes)] = jnp.ones((lanes,), dtype)
        return carry
      lax.fori_loop(0, _B, ones_row, 0)

      def batch(b, carry):
        for k in range(8):
          pltpu.async_copy(rows_v, acc.at[dst_v.at[b * 8 + k]], sem, add=True)
        for k in range(8):
          pltpu.make_async_copy(rows_v, acc.at[dst_v.at[b * 8 + k]], sem).wait()
        return carry
      lax.fori_loop(0, _RPW // 8, batch, 0)
    plsc.subcore_barrier()

    pltpu.sync_copy(acc.at[pl.ds(tid * _RPT, _RPT)],
                    out_hbm.at[cid, pl.ds(tid * _RPT, _RPT)])

  return functools.partial(
      pl.kernel,
      mesh=mesh,
      out_type=jax.ShapeDtypeStruct((_NC, _N, D), dtype),
      scratch_types=scratch,
      compiler_params=pltpu.CompilerParams(use_tc_tiling_on_sc=False),
  )(body)


def _row_spec(d):
  return pl.BlockSpec((_BLK, d), lambda i: (i, 0))


def _part_spec(d):
  return pl.BlockSpec((_NC, _BLK, d), lambda i: (0, i, 0))


def _full_spec(a, b):
  return pl.BlockSpec((a, b), lambda i: (0, 0))


def _tc0_body(x_r, we1, be1, wc, bc, wd1, bd1, wxb, bxb,
              enc_o, z_o, xbar_o):
  x = x_r[...]
  enc = jnp.maximum(jnp.dot(x, we1[...], preferred_element_type=F32) + be1[...], 0.0)
  enc_o[...] = enc
  z = jnp.dot(enc, wc[...], preferred_element_type=F32) + bc[...]
  z_o[...] = z
  dec = jnp.maximum(jnp.dot(z, wd1[...], preferred_element_type=F32) + bd1[...], 0.0)
  xbar_o[...] = jnp.dot(dec, wxb[...], preferred_element_type=F32) + bxb[...]


def _tc0(x, W_enc1, b_enc1, W_class, b_class, W_dec1, b_dec1, W_xbar, b_xbar):
  return pl.pallas_call(
      _tc0_body,
      grid=(_N // _BLK,),
      in_specs=[
          _row_spec(128),
          _full_spec(128, 256), _full_spec(1, 256),
          _full_spec(256, 64), _full_spec(1, 64),
          _full_spec(64, 256), _full_spec(1, 256),
          _full_spec(256, 128), _full_spec(1, 128),
      ],
      out_specs=[_row_spec(256), _row_spec(64), _row_spec(128)],
      out_shape=[
          jax.ShapeDtypeStruct((_N, 256), F32),
          jax.ShapeDtypeStruct((_N, 64), F32),
          jax.ShapeDtypeStruct((_N, 128), F32),
      ],
  )(x, W_enc1, b_enc1, W_class, b_class, W_dec1, b_dec1, W_xbar, b_xbar)


def _tc1_body(x_r, dg_r, xs_o, dis_o):
  x = x_r[...]
  deg = dg_r[0, :, 0:1] + dg_r[1, :, 0:1] + 1.0
  dis = lax.rsqrt(deg)
  dis_o[...] = dis
  xs_o[...] = (x * dis).astype(jnp.bfloat16)


def _tc1(x, degp):
  return pl.pallas_call(
      _tc1_body,
      grid=(_N // _BLK,),
      in_specs=[_row_spec(128), _part_spec(16)],
      out_specs=[_row_spec(128), _row_spec(1)],
      out_shape=[
          jax.ShapeDtypeStruct((_N, 128), jnp.bfloat16),
          jax.ShapeDtypeStruct((_N, 1), F32),
      ],
  )(x, degp)


def _tc2_body(acc_r, x_r, dis_r, enc_r, z_r, wg1, bg1, wg2, wpnd,
              m2_o, m3_o, mc_o):
  dis = dis_r[...]
  a = acc_r[0].astype(F32) + acc_r[1].astype(F32)
  out1 = dis * a + (dis * dis) * x_r[...]
  h1 = jnp.dot(out1, wg1[...], preferred_element_type=F32) + bg1[...]
  u = 0.5 * h1 + 0.5 * enc_r[...]
  m2 = jnp.dot(u, wg2[...], preferred_element_type=F32)
  m3 = jnp.dot(z_r[...], wpnd[...], preferred_element_type=F32)
  m2_o[...] = m2
  m3_o[...] = m3
  mc_o[...] = jnp.concatenate(
      [m2 * dis, m3 * dis, jnp.zeros((_BLK, 15), F32)], axis=1
  ).astype(jnp.bfloat16)


def _tc2(acc1, x, dis, enc, z, W_g1, b_g1, W_g2, W_pnd):
  return pl.pallas_call(
      _tc2_body,
      grid=(_N // _BLK,),
      in_specs=[
          _part_spec(128), _row_spec(128), _row_spec(1),
          _row_spec(256), _row_spec(64),
          _full_spec(128, 256), _full_spec(1, 256),
          _full_spec(256, 16), _full_spec(64, 1),
      ],
      out_specs=[_row_spec(16), _row_spec(1), _row_spec(32)],
      out_shape=[
          jax.ShapeDtypeStruct((_N, 16), F32),
          jax.ShapeDtypeStruct((_N, 1), F32),
          jax.ShapeDtypeStruct((_N, 32), jnp.bfloat16),
      ],
  )(acc1, x, dis, enc, z, W_g1, b_g1, W_g2, W_pnd)


def _tc3_body(acc_r, m2_r, m3_r, dis_r, bg2, bpnd, pred_o, hp_o):
  dis = dis_r[...]
  d2 = dis * dis
  a = acc_r[0].astype(F32) + acc_r[1].astype(F32)
  h2 = dis * a[:, 0:16] + d2 * m2_r[...] + bg2[...]
  mx = jnp.max(h2, axis=1, keepdims=True)
  e = h2 - mx
  lse = jnp.log(jnp.sum(jnp.exp(e), axis=1, keepdims=True))
  pred_o[...] = e - lse
  hp_o[...] = dis * a[:, 16:17] + d2 * m3_r[...] + bpnd[...]


def _tc3(acc2, m2, m3, dis, b_g2, b_pnd):
  return pl.pallas_call(
      _tc3_body,
      grid=(_N // _BLK,),
      in_specs=[
          _part_spec(32), _row_spec(16), _row_spec(1), _row_spec(1),
          _full_spec(1, 16), _full_spec(1, 1),
      ],
      out_specs=[_row_spec(16), _row_spec(1)],
      out_shape=[
          jax.ShapeDtypeStruct((_N, 16), F32),
          jax.ShapeDtypeStruct((_N, 1), F32),
      ],
  )(acc2, m2, m3, dis, b_g2, b_pnd)


def kernel(x, edge_index, W_enc1, b_enc1, W_class, b_class, W_dec1, b_dec1,
           W_xbar, b_xbar, W_g1, b_g1, W_g2, b_g2, W_pnd, b_pnd):
  ei = edge_index.astype(jnp.int32)
  src2d = ei[0].reshape(_E // _B, _B)
  dst2d = ei[1].reshape(_E // _B, _B)

  degp = _make_agg(16, gather=False)(dst2d)
  enc, z, x_bar = _tc0(
      x, W_enc1, b_enc1.reshape(1, -1), W_class, b_class.reshape(1, -1),
      W_dec1, b_dec1.reshape(1, -1), W_xbar, b_xbar.reshape(1, -1))
  xs, dis = _tc1(x, degp)
  acc1 = _make_agg(128, gather=True, dtype=jnp.bfloat16)(xs, src2d, dst2d)
  m2, m3, mc = _tc2(acc1, x, dis, enc, z, W_g1, b_g1.reshape(1, -1), W_g2, W_pnd)
  acc2 = _make_agg(32, gather=True, dtype=jnp.bfloat16)(mc, src2d, dst2d)
  predict, h_pred_nd = _tc3(acc2, m2, m3, dis,
                            b_g2.reshape(1, -1), b_pnd.reshape(1, -1))
  return (x_bar, predict, h_pred_nd)


# speedup vs baseline: 46.4973x; 1.0196x over previous
import functools

import jax
import jax.numpy as jnp
from jax import lax
from jax.experimental import pallas as pl
from jax.experimental.pallas import tpu as pltpu
from jax.experimental.pallas import tpu_sc as plsc

F32 = jnp.float32

_N = 10000
_E = 320000
_B = 125
_NC = 2
_NS = 16
_NW = _NC * _NS
_RPW = _E // (_B * _NW)
_RPT = _N // _NS
_BLK = 1000


@functools.cache
def _make_agg(D, gather, dtype=F32):
  lanes = 32 if dtype == jnp.bfloat16 else 16
  mesh = plsc.VectorSubcoreMesh(core_axis_name="c", subcore_axis_name="s")
  nsec = 1
  srpw = _RPW // nsec
  scratch = []
  nbuf = 8 if gather else 4
  if gather:
    scratch.append(pltpu.VMEM((srpw, _B), jnp.int32))
  scratch += [
      pltpu.VMEM((srpw, _B), jnp.int32),
  ]
  scratch += [pltpu.VMEM((_B, D), dtype)] * nbuf
  scratch += [pltpu.VMEM_SHARED((_N, D), dtype)]
  scratch += [pltpu.SemaphoreType.DMA] * (2 * nbuf)

  def body(*refs):
    if gather:
      (vals_hbm, src_hbm, dst_hbm, out_hbm, src_v, dst_v) = refs[:6]
      rest = refs[6:]
    else:
      (dst_hbm, out_hbm, dst_v) = refs[:3]
      rest = refs[3:]
    rows = rest[:nbuf]
    acc = rest[nbuf]
    gsem = rest[nbuf + 1:nbuf + 1 + nbuf]
    ssem = rest[nbuf + 1 + nbuf:]
    rows_v = rows[0]
    sem = gsem[0]
    cid = lax.axis_index("c")
    tid = lax.axis_index("s")
    wid = tid * _NC + cid

    def zero_row(r, carry):
      for j in range(D // lanes):
        rows_v[r, pl.ds(j * lanes, lanes)] = jnp.zeros((lanes,), dtype)
      return carry
    lax.fori_loop(0, _B, zero_row, 0)
    for k in range(_RPT // _B):
      pltpu.sync_copy(rows_v, acc.at[pl.ds(tid * _RPT + k * _B, _B)])
    plsc.subcore_barrier()

    if gather:
      for s in range(nsec):
        base = wid * _RPW + s * srpw
        pltpu.sync_copy(dst_hbm.at[pl.ds(base, srpw)], dst_v)
        pltpu.sync_copy(src_hbm.at[pl.ds(base, srpw)], src_v)
        for i in range(nbuf):
          pltpu.async_copy(vals_hbm.at[src_v.at[i]], rows[i], gsem[i])

        def chunkn(k, carry):
          c = k * nbuf
          for i in range(nbuf):
            pltpu.make_async_copy(
                vals_hbm.at[src_v.at[c + i]], rows[i], gsem[i]).wait()
            pltpu.async_copy(rows[i], acc.at[dst_v.at[c + i]], ssem[i],
                             add=True)
          for i in range(nbuf):
            pltpu.make_async_copy(
                rows[i], acc.at[dst_v.at[c + i]], ssem[i]).wait()

            @pl.when(c + nbuf + i < srpw)
            def _():
              pltpu.async_copy(
                  vals_hbm.at[src_v.at[c + nbuf + i]], rows[i], gsem[i])
          return carry
        lax.fori_loop(0, srpw // nbuf, chunkn, 0)
    else:
      pltpu.sync_copy(dst_hbm.at[pl.ds(wid * _RPW, _RPW)], dst_v)

      def ones_row(r, carry):
        for j in range(D // lanes):
          rows_v[r, pl.ds(j * lanes, lanes)] = jnp.ones((lanes,), dtype)
        return carry
      lax.fori_loop(0, _B, ones_row, 0)

      def batch(b, carry):
        for k in range(8):
          pltpu.async_copy(rows_v, acc.at[dst_v.at[b * 8 + k]], sem, add=True)
        for k in range(8):
          pltpu.make_async_copy(rows_v, acc.at[dst_v.at[b * 8 + k]], sem).wait()
        return carry
      lax.fori_loop(0, _RPW // 8, batch, 0)
    plsc.subcore_barrier()

    pltpu.sync_copy(acc.at[pl.ds(tid * _RPT, _RPT)],
                    out_hbm.at[cid, pl.ds(tid * _RPT, _RPT)])

  return functools.partial(
      pl.kernel,
      mesh=mesh,
      out_type=jax.ShapeDtypeStruct((_NC, _N, D), dtype),
      scratch_types=scratch,
      compiler_params=pltpu.CompilerParams(use_tc_tiling_on_sc=False),
  )(body)


def _row_spec(d):
  return pl.BlockSpec((_BLK, d), lambda i: (i, 0))


def _part_spec(d):
  return pl.BlockSpec((_NC, _BLK, d), lambda i: (0, i, 0))


def _full_spec(a, b):
  return pl.BlockSpec((a, b), lambda i: (0, 0))


def _tc0_body(x_r, we1, be1, wc, bc, wd1, bd1, wxb, bxb,
              enc_o, z_o, xbar_o):
  x = x_r[...]
  enc = jnp.maximum(jnp.dot(x, we1[...], preferred_element_type=F32) + be1[...], 0.0)
  enc_o[...] = enc
  z = jnp.dot(enc, wc[...], preferred_element_type=F32) + bc[...]
  z_o[...] = z
  dec = jnp.maximum(jnp.dot(z, wd1[...], preferred_element_type=F32) + bd1[...], 0.0)
  xbar_o[...] = jnp.dot(dec, wxb[...], preferred_element_type=F32) + bxb[...]


def _tc0(x, W_enc1, b_enc1, W_class, b_class, W_dec1, b_dec1, W_xbar, b_xbar):
  return pl.pallas_call(
      _tc0_body,
      grid=(_N // _BLK,),
      in_specs=[
          _row_spec(128),
          _full_spec(128, 256), _full_spec(1, 256),
          _full_spec(256, 64), _full_spec(1, 64),
          _full_spec(64, 256), _full_spec(1, 256),
          _full_spec(256, 128), _full_spec(1, 128),
      ],
      out_specs=[_row_spec(256), _row_spec(64), _row_spec(128)],
      out_shape=[
          jax.ShapeDtypeStruct((_N, 256), F32),
          jax.ShapeDtypeStruct((_N, 64), F32),
          jax.ShapeDtypeStruct((_N, 128), F32),
      ],
  )(x, W_enc1, b_enc1, W_class, b_class, W_dec1, b_dec1, W_xbar, b_xbar)


def _tc1_body(x_r, dg_r, xs_o, dis_o):
  x = x_r[...]
  deg = dg_r[0, :, 0:1] + dg_r[1, :, 0:1] + 1.0
  dis = lax.rsqrt(deg)
  dis_o[...] = dis
  xs_o[...] = (x * dis).astype(jnp.bfloat16)


def _tc1(x, degp):
  return pl.pallas_call(
      _tc1_body,
      grid=(_N // _BLK,),
      in_specs=[_row_spec(128), _part_spec(16)],
      out_specs=[_row_spec(128), _row_spec(1)],
      out_shape=[
          jax.ShapeDtypeStruct((_N, 128), jnp.bfloat16),
          jax.ShapeDtypeStruct((_N, 1), F32),
      ],
  )(x, degp)


def _tc2_body(acc_r, x_r, dis_r, enc_r, z_r, wg1, bg1, wg2, wpnd,
              m2_o, m3_o, mc_o):
  dis = dis_r[...]
  a = acc_r[0].astype(F32) + acc_r[1].astype(F32)
  out1 = dis * a + (dis * dis) * x_r[...]
  h1 = jnp.dot(out1, wg1[...], preferred_element_type=F32) + bg1[...]
  u = 0.5 * h1 + 0.5 * enc_r[...]
  m2 = jnp.dot(u, wg2[...], preferred_element_type=F32)
  m3 = jnp.dot(z_r[...], wpnd[...], preferred_element_type=F32)
  m2_o[...] = m2
  m3_o[...] = m3
  mc_o[...] = jnp.concatenate(
      [m2 * dis, m3 * dis, jnp.zeros((_BLK, 15), F32)], axis=1
  ).astype(jnp.bfloat16)


def _tc2(acc1, x, dis, enc, z, W_g1, b_g1, W_g2, W_pnd):
  return pl.pallas_call(
      _tc2_body,
      grid=(_N // _BLK,),
      in_specs=[
          _part_spec(128), _row_spec(128), _row_spec(1),
          _row_spec(256), _row_spec(64),
          _full_spec(128, 256), _full_spec(1, 256),
          _full_spec(256, 16), _full_spec(64, 1),
      ],
      out_specs=[_row_spec(16), _row_spec(1), _row_spec(32)],
      out_shape=[
          jax.ShapeDtypeStruct((_N, 16), F32),
          jax.ShapeDtypeStruct((_N, 1), F32),
          jax.ShapeDtypeStruct((_N, 32), jnp.bfloat16),
      ],
  )(acc1, x, dis, enc, z, W_g1, b_g1, W_g2, W_pnd)


def _tc3_body(acc_r, m2_r, m3_r, dis_r, bg2, bpnd, pred_o, hp_o):
  dis = dis_r[...]
  d2 = dis * dis
  a = acc_r[0].astype(F32) + acc_r[1].astype(F32)
  h2 = dis * a[:, 0:16] + d2 * m2_r[...] + bg2[...]
  mx = jnp.max(h2, axis=1, keepdims=True)
  e = h2 - mx
  lse = jnp.log(jnp.sum(jnp.exp(e), axis=1, keepdims=True))
  pred_o[...] = e - lse
  hp_o[...] = dis * a[:, 16:17] + d2 * m3_r[...] + bpnd[...]


def _tc3(acc2, m2, m3, dis, b_g2, b_pnd):
  return pl.pallas_call(
      _tc3_body,
      grid=(_N // _BLK,),
      in_specs=[
          _part_spec(32), _row_spec(16), _row_spec(1), _row_spec(1),
          _full_spec(1, 16), _full_spec(1, 1),
      ],
      out_specs=[_row_spec(16), _row_spec(1)],
      out_shape=[
          jax.ShapeDtypeStruct((_N, 16), F32),
          jax.ShapeDtypeStruct((_N, 1), F32),
      ],
  )(acc2, m2, m3, dis, b_g2, b_pnd)


def kernel(x, edge_index, W_enc1, b_enc1, W_class, b_class, W_dec1, b_dec1,
           W_xbar, b_xbar, W_g1, b_g1, W_g2, b_g2, W_pnd, b_pnd):
  ei = edge_index.astype(jnp.int32)
  src2d = ei[0].reshape(_E // _B, _B)
  dst2d = ei[1].reshape(_E // _B, _B)

  degp = _make_agg(16, gather=False)(dst2d)
  enc, z, x_bar = _tc0(
      x, W_enc1, b_enc1.reshape(1, -1), W_class, b_class.reshape(1, -1),
      W_dec1, b_dec1.reshape(1, -1), W_xbar, b_xbar.reshape(1, -1))
  xs, dis = _tc1(x, degp)
  acc1 = _make_agg(128, gather=True, dtype=jnp.bfloat16)(xs, src2d, dst2d)
  m2, m3, mc = _tc2(acc1, x, dis, enc, z, W_g1, b_g1.reshape(1, -1), W_g2, W_pnd)
  acc2 = _make_agg(32, gather=True, dtype=jnp.bfloat16)(mc, src2d, dst2d)
  predict, h_pred_nd = _tc3(acc2, m2, m3, dis,
                            b_g2.reshape(1, -1), b_pnd.reshape(1, -1))
  return (x_bar, predict, h_pred_nd)
